# Initial kernel scaffold; baseline (speedup 1.0000x reference)
import jax
import jax.numpy as jnp
from jax.experimental import pallas as pl


def kernel(x, point_positions, neuron_pad_mask, spike_mask, Wq, Wk, Wv, Wo, rms_w, rope_dirs, rope_freqs, rff_Omega, posC_W, pos_head_gain):
    raise NotImplementedError("write your pallas kernel here")


# speedup vs baseline: 2.0081x; 2.0081x over previous
import numpy as np
import jax
import jax.numpy as jnp
from jax.experimental import pallas as pl
from jax.experimental.pallas import tpu as pltpu

_H = 8
_DH = 64
_M = 16
_DP = 16
_POS_SCALE = 0.1
_EPS = 1e-6
_NEG = -1e30


def _attn_kernel(x_ref, wq_ref, wk_ref, wv_ref, wo_ref,
                 a_ref, bl_ref, br_ref, tq_ref, tk_ref,
                 bias_ref, omask_ref, out_ref):
    f32 = jnp.float32
    xb = x_ref[0, 0]
    r = jax.lax.rsqrt(jnp.mean(xb * xb, axis=-1, keepdims=True) + _EPS)
    xn = xb * r
    q0 = jnp.dot(xn, wq_ref[...], preferred_element_type=f32)
    k0 = jnp.dot(xn, wk_ref[...], preferred_element_type=f32)
    v = jnp.dot(xn, wv_ref[...], preferred_element_type=f32)

    def roll_l(u):
        return jnp.concatenate([u[:, _M:], u[:, :_M]], axis=1)

    def roll_r(u):
        return jnp.concatenate([u[:, -_M:], u[:, :-_M]], axis=1)

    A = a_ref[0]
    Bl = bl_ref[0]
    Br = br_ref[0]
    q = q0 * A + roll_l(q0) * Bl + roll_r(q0) * Br + tq_ref[0]
    k = k0 * A + roll_l(k0) * Bl + roll_r(k0) * Br + tk_ref[0]

    bias = bias_ref[0, 0]
    scale = 1.0 / np.sqrt(_DH)
    outs = []
    for h in range(_H):
        sl = slice(h * _DH, (h + 1) * _DH)
        qh = q[:, sl]
        kh = k[:, sl]
        s = jax.lax.dot_general(qh, kh, (((1,), (1,)), ((), ())),
                                preferred_element_type=f32) * scale + bias
        m = jnp.max(s, axis=-1, keepdims=True)
        e = jnp.exp(s - m)
        p = e / jnp.sum(e, axis=-1, keepdims=True)
        outs.append(jnp.dot(p, v[:, sl], preferred_element_type=f32))
    o = jnp.concatenate(outs, axis=1)
    y = jnp.dot(o, wo_ref[...], preferred_element_type=f32)
    out_ref[0, 0] = y * omask_ref[0, 0]


def kernel(x, point_positions, neuron_pad_mask, spike_mask, Wq, Wk, Wv, Wo,
           rms_w, rope_dirs, rope_freqs, rff_Omega, posC_W, pos_head_gain):
    f32 = jnp.float32
    B, T, N, D = x.shape
    pp = point_positions
    angles = jnp.einsum('bnd,fd->bnf', pp, rope_dirs) * rope_freqs
    th = angles[..., :_M]
    c = jnp.cos(th)
    s = jnp.sin(th)
    proj = jnp.einsum('bnd,md->bnm', pp, rff_Omega)
    phi = jnp.concatenate([jnp.cos(proj), jnp.sin(proj)], axis=-1)
    pos_feat = jnp.einsum('bnm,pm->bnp', phi, posC_W)
    z = jnp.zeros_like(c)
    one = jnp.ones_like(c)
    a_blk = jnp.concatenate([c, c - s * s, one, z], axis=-1)
    bl_blk = jnp.concatenate([-s, z, z, z], axis=-1)
    br_blk = jnp.concatenate([z, c * s, z, z], axis=-1)
    A = jnp.tile(a_blk, (1, 1, _H))
    Bl = jnp.tile(bl_blk, (1, 1, _H))
    Br = jnp.tile(br_blk, (1, 1, _H))
    z48 = jnp.zeros(c.shape[:2] + (_DH - _DP,), f32)
    Tk = jnp.tile(jnp.concatenate([z48, pos_feat], axis=-1), (1, 1, _H))
    Tq = jnp.concatenate(
        [jnp.concatenate(
            [z48, _POS_SCALE * pos_head_gain[h][None, None, :] * pos_feat],
            axis=-1) for h in range(_H)],
        axis=-1)
    perm = np.concatenate([
        h * _DH + np.concatenate([np.arange(0, 2 * _M, 2),
                                  np.arange(1, 2 * _M, 2),
                                  np.arange(2 * _M, _DH)])
        for h in range(_H)])
    Wq_p = (Wq * rms_w[None, :]).T[:, perm]
    Wk_p = (Wk * rms_w[None, :]).T[:, perm]
    Wv_s = (Wv * rms_w[None, :]).T
    Wo = Wo.T
    valid = neuron_pad_mask != 0
    spk = (spike_mask != 0) & valid[:, None, :]
    bias = jnp.where(spk, 0.0, _NEG).astype(f32)[:, :, None, :]
    has_key = jnp.any(spk, axis=-1)
    omask = (valid[:, None, :] & has_key[:, :, None]).astype(f32)[..., None]

    out = pl.pallas_call(
        _attn_kernel,
        grid=(B, T),
        in_specs=[
            pl.BlockSpec((1, 1, N, D), lambda b, t: (b, t, 0, 0)),
            pl.BlockSpec((D, D), lambda b, t: (0, 0)),
            pl.BlockSpec((D, D), lambda b, t: (0, 0)),
            pl.BlockSpec((D, D), lambda b, t: (0, 0)),
            pl.BlockSpec((D, D), lambda b, t: (0, 0)),
            pl.BlockSpec((1, N, D), lambda b, t: (b, 0, 0)),
            pl.BlockSpec((1, N, D), lambda b, t: (b, 0, 0)),
            pl.BlockSpec((1, N, D), lambda b, t: (b, 0, 0)),
            pl.BlockSpec((1, N, D), lambda b, t: (b, 0, 0)),
            pl.BlockSpec((1, N, D), lambda b, t: (b, 0, 0)),
            pl.BlockSpec((1, 1, 1, N), lambda b, t: (b, t, 0, 0)),
            pl.BlockSpec((1, 1, N, 1), lambda b, t: (b, t, 0, 0)),
        ],
        out_specs=pl.BlockSpec((1, 1, N, D), lambda b, t: (b, t, 0, 0)),
        out_shape=jax.ShapeDtypeStruct((B, T, N, D), f32),
        compiler_params=pltpu.CompilerParams(
            dimension_semantics=("parallel", "parallel")),
    )(x, Wq_p, Wk_p, Wv_s, Wo, A, Bl, Br, Tq, Tk, bias, omask)
    return out


# speedup vs baseline: 2.1237x; 1.0576x over previous
import numpy as np
import jax
import jax.numpy as jnp
from jax.experimental import pallas as pl
from jax.experimental.pallas import tpu as pltpu

_H = 8
_DH = 64
_M = 16
_DP = 16
_POS_SCALE = 0.1
_EPS = 1e-6
_NEG = -1e30


def _attn_kernel(x_ref, wq_ref, wk_ref, wv_ref, wo_ref,
                 a_ref, bl_ref, br_ref, tq_ref, tk_ref,
                 bias_ref, omask_ref, out_ref):
    f32 = jnp.float32
    xb = x_ref[0, 0]
    r = jax.lax.rsqrt(jnp.mean(xb * xb, axis=-1, keepdims=True) + _EPS)
    xn = xb * r
    q0 = jnp.dot(xn, wq_ref[...], preferred_element_type=f32)
    k0 = jnp.dot(xn, wk_ref[...], preferred_element_type=f32)
    v = jnp.dot(xn, wv_ref[...], preferred_element_type=f32)

    def roll_l(u):
        return jnp.concatenate([u[:, _M:], u[:, :_M]], axis=1)

    def roll_r(u):
        return jnp.concatenate([u[:, -_M:], u[:, :-_M]], axis=1)

    A = a_ref[0]
    Bl = bl_ref[0]
    Br = br_ref[0]
    q = q0 * A + roll_l(q0) * Bl + roll_r(q0) * Br + tq_ref[0]
    k = k0 * A + roll_l(k0) * Bl + roll_r(k0) * Br + tk_ref[0]

    bias = bias_ref[0, 0]
    scale = 1.0 / np.sqrt(_DH)
    outs = []
    for h in range(_H):
        sl = slice(h * _DH, (h + 1) * _DH)
        qh = q[:, sl]
        kh = k[:, sl]
        s = jax.lax.dot_general(qh, kh, (((1,), (1,)), ((), ())),
                                preferred_element_type=f32) * scale + bias
        m = jnp.max(s, axis=-1, keepdims=True)
        e = jnp.exp(s - m)
        rs = 1.0 / jnp.sum(e, axis=-1, keepdims=True)
        oh = jnp.dot(e, v[:, sl], preferred_element_type=f32)
        outs.append(oh * rs)
    o = jnp.concatenate(outs, axis=1)
    y = jnp.dot(o, wo_ref[...], preferred_element_type=f32)
    out_ref[0, 0] = y * omask_ref[0, 0]


def kernel(x, point_positions, neuron_pad_mask, spike_mask, Wq, Wk, Wv, Wo,
           rms_w, rope_dirs, rope_freqs, rff_Omega, posC_W, pos_head_gain):
    f32 = jnp.float32
    B, T, N, D = x.shape
    pp = point_positions
    angles = jnp.einsum('bnd,fd->bnf', pp, rope_dirs) * rope_freqs
    th = angles[..., :_M]
    c = jnp.cos(th)
    s = jnp.sin(th)
    proj = jnp.einsum('bnd,md->bnm', pp, rff_Omega)
    phi = jnp.concatenate([jnp.cos(proj), jnp.sin(proj)], axis=-1)
    pos_feat = jnp.einsum('bnm,pm->bnp', phi, posC_W)
    z = jnp.zeros_like(c)
    one = jnp.ones_like(c)
    a_blk = jnp.concatenate([c, c - s * s, one, z], axis=-1)
    bl_blk = jnp.concatenate([-s, z, z, z], axis=-1)
    br_blk = jnp.concatenate([z, c * s, z, z], axis=-1)
    A = jnp.tile(a_blk, (1, 1, _H))
    Bl = jnp.tile(bl_blk, (1, 1, _H))
    Br = jnp.tile(br_blk, (1, 1, _H))
    z48 = jnp.zeros(c.shape[:2] + (_DH - _DP,), f32)
    Tk = jnp.tile(jnp.concatenate([z48, pos_feat], axis=-1), (1, 1, _H))
    Tq = jnp.concatenate(
        [jnp.concatenate(
            [z48, _POS_SCALE * pos_head_gain[h][None, None, :] * pos_feat],
            axis=-1) for h in range(_H)],
        axis=-1)
    perm = np.concatenate([
        h * _DH + np.concatenate([np.arange(0, 2 * _M, 2),
                                  np.arange(1, 2 * _M, 2),
                                  np.arange(2 * _M, _DH)])
        for h in range(_H)])
    Wq_p = (Wq * rms_w[None, :]).T[:, perm]
    Wk_p = (Wk * rms_w[None, :]).T[:, perm]
    Wv_s = (Wv * rms_w[None, :]).T
    Wo = Wo.T
    valid = neuron_pad_mask != 0
    spk = (spike_mask != 0) & valid[:, None, :]
    bias = jnp.where(spk, 0.0, _NEG).astype(f32)[:, :, None, :]
    has_key = jnp.any(spk, axis=-1)
    omask = (valid[:, None, :] & has_key[:, :, None]).astype(f32)[..., None]

    out = pl.pallas_call(
        _attn_kernel,
        grid=(B, T),
        in_specs=[
            pl.BlockSpec((1, 1, N, D), lambda b, t: (b, t, 0, 0)),
            pl.BlockSpec((D, D), lambda b, t: (0, 0)),
            pl.BlockSpec((D, D), lambda b, t: (0, 0)),
            pl.BlockSpec((D, D), lambda b, t: (0, 0)),
            pl.BlockSpec((D, D), lambda b, t: (0, 0)),
            pl.BlockSpec((1, N, D), lambda b, t: (b, 0, 0)),
            pl.BlockSpec((1, N, D), lambda b, t: (b, 0, 0)),
            pl.BlockSpec((1, N, D), lambda b, t: (b, 0, 0)),
            pl.BlockSpec((1, N, D), lambda b, t: (b, 0, 0)),
            pl.BlockSpec((1, N, D), lambda b, t: (b, 0, 0)),
            pl.BlockSpec((1, 1, 1, N), lambda b, t: (b, t, 0, 0)),
            pl.BlockSpec((1, 1, N, 1), lambda b, t: (b, t, 0, 0)),
        ],
        out_specs=pl.BlockSpec((1, 1, N, D), lambda b, t: (b, t, 0, 0)),
        out_shape=jax.ShapeDtypeStruct((B, T, N, D), f32),
        compiler_params=pltpu.CompilerParams(
            dimension_semantics=("parallel", "parallel")),
    )(x, Wq_p, Wk_p, Wv_s, Wo, A, Bl, Br, Tq, Tk, bias, omask)
    return out


# speedup vs baseline: 2.3180x; 1.0915x over previous
import numpy as np
import jax
import jax.numpy as jnp
from jax.experimental import pallas as pl
from jax.experimental.pallas import tpu as pltpu

_H = 8
_DH = 64
_M = 16
_DP = 16
_POS_SCALE = 0.1
_EPS = 1e-6


def _attn_kernel(x_ref, wq_ref, wk_ref, wv_ref, wo_ref,
                 pp_ref, rdt_ref, rot_ref, pwt_ref, gains_ref,
                 spk_ref, valid_ref, out_ref,
                 a_s, bl_s, br_s, tq_s, tk_s):
    f32 = jnp.float32
    t = pl.program_id(1)

    @pl.when(t == 0)
    def _build_planes():
        pp = pp_ref[0]
        px, py, pz = pp[:, 0:1], pp[:, 1:2], pp[:, 2:3]
        th = px * rdt_ref[0:1, :] + py * rdt_ref[1:2, :] + pz * rdt_ref[2:3, :]
        c = jnp.cos(th)
        s = jnp.sin(th)
        proj = (px * rot_ref[0:1, :] + py * rot_ref[1:2, :]
                + pz * rot_ref[2:3, :])
        phi = jnp.concatenate([jnp.cos(proj), jnp.sin(proj)], axis=1)
        pf = jnp.dot(phi, pwt_ref[...], preferred_element_type=f32)
        one = jnp.ones_like(c)
        zr = jnp.zeros_like(c)
        a_s[...] = jnp.concatenate([c, c - s * s, one, zr] * _H, axis=1)
        bl_s[...] = jnp.concatenate([-s, zr, zr, zr] * _H, axis=1)
        br_s[...] = jnp.concatenate([zr, c * s, zr, zr] * _H, axis=1)
        tq_parts = []
        for h in range(_H):
            tq_parts.extend([zr, zr, zr, gains_ref[h:h + 1, :] * pf])
        tq_s[...] = jnp.concatenate(tq_parts, axis=1)
        tk_s[...] = jnp.concatenate([zr, zr, zr, pf] * _H, axis=1)

    xb = x_ref[0, 0]
    r = jax.lax.rsqrt(jnp.mean(xb * xb, axis=-1, keepdims=True) + _EPS)
    xn = xb * r
    q0 = jnp.dot(xn, wq_ref[...], preferred_element_type=f32)
    k0 = jnp.dot(xn, wk_ref[...], preferred_element_type=f32)
    v = jnp.dot(xn, wv_ref[...], preferred_element_type=f32)

    def roll_l(u):
        return jnp.concatenate([u[:, _M:], u[:, :_M]], axis=1)

    def roll_r(u):
        return jnp.concatenate([u[:, -_M:], u[:, :-_M]], axis=1)

    A = a_s[...]
    Bl = bl_s[...]
    Br = br_s[...]
    q = q0 * A + roll_l(q0) * Bl + roll_r(q0) * Br + tq_s[...]
    k = k0 * A + roll_l(k0) * Bl + roll_r(k0) * Br + tk_s[...]

    spk_row = spk_ref[0, 0]
    bias = (spk_row - 1.0) * 1e30
    has_key = jnp.max(spk_row)
    scale = 1.0 / np.sqrt(_DH)
    outs = []
    for h in range(_H):
        sl = slice(h * _DH, (h + 1) * _DH)
        sc = jax.lax.dot_general(q[:, sl], k[:, sl],
                                 (((1,), (1,)), ((), ())),
                                 preferred_element_type=f32) * scale + bias
        m = jnp.max(sc, axis=-1, keepdims=True)
        e = jnp.exp(sc - m)
        rs = 1.0 / jnp.sum(e, axis=-1, keepdims=True)
        oh = jnp.dot(e, v[:, sl], preferred_element_type=f32)
        outs.append(oh * rs)
    o = jnp.concatenate(outs, axis=1)
    y = jnp.dot(o, wo_ref[...], preferred_element_type=f32)
    out_ref[0, 0] = y * (valid_ref[0] * has_key)


def kernel(x, point_positions, neuron_pad_mask, spike_mask, Wq, Wk, Wv, Wo,
           rms_w, rope_dirs, rope_freqs, rff_Omega, posC_W, pos_head_gain):
    f32 = jnp.float32
    B, T, N, D = x.shape
    perm = np.concatenate([
        h * _DH + np.concatenate([np.arange(0, 2 * _M, 2),
                                  np.arange(1, 2 * _M, 2),
                                  np.arange(2 * _M, _DH)])
        for h in range(_H)])
    Wq_p = (Wq * rms_w[None, :]).T[:, perm]
    Wk_p = (Wk * rms_w[None, :]).T[:, perm]
    Wv_s = (Wv * rms_w[None, :]).T
    WoT = Wo.T
    pp_pad = jnp.pad(point_positions, ((0, 0), (0, 0), (0, 5)))
    rdt = jnp.pad((rope_dirs[:_M] * rope_freqs[:_M, None]).T, ((0, 5), (0, 0)))
    rot = jnp.pad(rff_Omega.T, ((0, 5), (0, 0)))
    pwt = posC_W.T
    gains = _POS_SCALE * pos_head_gain
    valid = neuron_pad_mask != 0
    spkf = ((spike_mask != 0) & valid[:, None, :]).astype(f32)[:, :, None, :]
    validf = valid.astype(f32)[:, :, None]

    out = pl.pallas_call(
        _attn_kernel,
        grid=(B, T),
        in_specs=[
            pl.BlockSpec((1, 1, N, D), lambda b, t: (b, t, 0, 0)),
            pl.BlockSpec((D, D), lambda b, t: (0, 0)),
            pl.BlockSpec((D, D), lambda b, t: (0, 0)),
            pl.BlockSpec((D, D), lambda b, t: (0, 0)),
            pl.BlockSpec((D, D), lambda b, t: (0, 0)),
            pl.BlockSpec((1, N, 8), lambda b, t: (b, 0, 0)),
            pl.BlockSpec((8, _M), lambda b, t: (0, 0)),
            pl.BlockSpec((8, 2 * _M), lambda b, t: (0, 0)),
            pl.BlockSpec((4 * _M, _DP), lambda b, t: (0, 0)),
            pl.BlockSpec((_H, _DP), lambda b, t: (0, 0)),
            pl.BlockSpec((1, 1, 1, N), lambda b, t: (b, t, 0, 0)),
            pl.BlockSpec((1, N, 1), lambda b, t: (b, 0, 0)),
        ],
        out_specs=pl.BlockSpec((1, 1, N, D), lambda b, t: (b, t, 0, 0)),
        out_shape=jax.ShapeDtypeStruct((B, T, N, D), f32),
        scratch_shapes=[pltpu.VMEM((N, D), f32)] * 5,
        compiler_params=pltpu.CompilerParams(
            dimension_semantics=("parallel", "arbitrary")),
    )(x, Wq_p, Wk_p, Wv_s, WoT, pp_pad, rdt, rot, pwt, gains, spkf, validf)
    return out
